# baseline (speedup 1.0000x reference)
import functools

import jax
import jax.numpy as jnp
from jax import lax
from jax.experimental import pallas as pl
from jax.experimental.pallas import tpu as pltpu
from jax.experimental.pallas import tpu_sc as plsc

_N = 4096
_L = 200
_NW = 32
_RW = _N // _NW
_NG = _RW // 16


def _sc_sort_key(t):
    s = lax.bitcast_convert_type(t, jnp.int32)
    m = lax.shift_right_arithmetic(s, 31)
    return lax.bitcast_convert_type(s ^ (m | jnp.int32(-2147483648)),
                                    jnp.uint32)


def _sc_log(x):
    i = lax.bitcast_convert_type(x, jnp.int32)
    y = (i.astype(jnp.float32) - jnp.float32(1064866805.0)) \
        * jnp.float32(8.262958405176314e-08)
    for _ in range(3):
        y = y + x * jnp.exp(-y) - jnp.float32(1.0)
    return y


def _sc_body(pt_hbm, tt_hbm, out_hbm, p_v, t_v, kap_v, e_v, acc_v):
    wid = lax.axis_index("s") * 2 + lax.axis_index("c")

    acc_v[0] = jnp.zeros((16,), jnp.float32)
    pltpu.sync_copy(pt_hbm.at[wid], p_v)
    pltpu.sync_copy(tt_hbm.at[wid], t_v)

    def group(g, carry):
        goff = g * 16

        def stats(l, c):
            mx, sp = c
            row = p_v[l, pl.ds(goff, 16)]
            return jnp.maximum(mx, row), sp + row

        m, sum_p = lax.fori_loop(
            0, _L, stats,
            (jnp.full((16,), -jnp.inf, jnp.float32),
             jnp.zeros((16,), jnp.float32)))

        def prep(l, c):
            kap_v[l, pl.ds(goff, 16)] = _sc_sort_key(t_v[l, pl.ds(goff, 16)])
            e_v[l, pl.ds(goff, 16)] = jnp.exp(p_v[l, pl.ds(goff, 16)] - m)
            return c

        lax.fori_loop(0, _L, prep, 0)

        def query(k, logsum):
            kb = kap_v[k, pl.ds(goff, 16)] + jnp.uint32(1)
            kap_v[k, pl.ds(goff, 16)] = kb
            kap_k = kb - jnp.uint32(1)
            tsum = jnp.zeros((16,), jnp.float32)
            for l in range(_L):
                tsum = tsum + jnp.where(
                    kap_v[l, pl.ds(goff, 16)] > kap_k,
                    e_v[l, pl.ds(goff, 16)], 0.0)
            return logsum + _sc_log(tsum)

        logsum = lax.fori_loop(0, _L, query, jnp.zeros((16,), jnp.float32))

        acc_v[0] = acc_v[0] + (-sum_p + logsum + jnp.float32(_L) * m)
        return carry

    lax.fori_loop(0, _NG, group, 0)
    pltpu.sync_copy(acc_v, out_hbm.at[pl.ds(wid, 1), :])


@jax.jit
def kernel(y_pred, y_true):
    pt = y_pred.reshape(_NW, _RW, _L).transpose(0, 2, 1)
    tt = y_true.reshape(_NW, _RW, _L).transpose(0, 2, 1)
    mesh = plsc.VectorSubcoreMesh(core_axis_name="c", subcore_axis_name="s")
    partials = pl.kernel(
        _sc_body,
        mesh=mesh,
        out_type=jax.ShapeDtypeStruct((_NW, 16), jnp.float32),
        scratch_types=[
            pltpu.VMEM((_L, _RW), jnp.float32),
            pltpu.VMEM((_L, _RW), jnp.float32),
            pltpu.VMEM((_L, _RW), jnp.uint32),
            pltpu.VMEM((_L, _RW), jnp.float32),
            pltpu.VMEM((1, 16), jnp.float32),
        ],
    )(pt, tt)
    return jnp.sum(partials) * jnp.float32(1.0 / _N)


# speedup vs baseline: 5.5763x; 5.5763x over previous
import functools

import jax
import jax.numpy as jnp
from jax import lax
from jax.experimental import pallas as pl
from jax.experimental.pallas import tpu as pltpu
from jax.experimental.pallas import tpu_sc as plsc

_N = 4096
_L = 200
_C = 256
_G = 8
_NT = _L // _G
_TCB = 14
_SCR = _N - _TCB * _C
_NW = 32
_NSLAB = _SCR // 128


def _sort_key(t):
    s = jax.lax.bitcast_convert_type(t, jnp.int32)
    m = jax.lax.shift_right_arithmetic(s, 31)
    return jax.lax.bitcast_convert_type(s ^ (m | jnp.int32(-2147483648)),
                                        jnp.uint32)


def _body(pt_ref, tt_ref, out_ref, kap_ref, e_ref, tmat_ref):
    i = pl.program_id(0)
    p = pt_ref[...]
    t = tt_ref[...]
    m = jnp.max(p, axis=0, keepdims=True)
    e_ref[: _L, :] = jnp.exp(p - m)
    sum_p = jnp.sum(p, axis=0, keepdims=True)
    kap_ref[: _L, :] = _sort_key(t)

    iota8 = jax.lax.broadcasted_iota(jnp.uint32, (_G, _C), 0)

    def group(g, carry):
        base = g * _G
        kq = kap_ref[pl.ds(base, _G), :]
        eq8 = e_ref[pl.ds(base, _G), :]
        kqb = [jnp.broadcast_to(kq[j:j + 1, :], (_G, _C)) for j in range(_G)]
        accs = [jnp.zeros((_G, _C), jnp.float32) for _ in range(_G)]
        for tile in range(_NT):
            lhs = kap_ref[tile * _G:(tile + 1) * _G, :]
            et = e_ref[tile * _G:(tile + 1) * _G, :]
            for j in range(_G):
                accs[j] = accs[j] + jnp.where(lhs > kqb[j], et, 0.0)
        tks = []
        for j in range(_G):
            corr = jnp.where((kq == kqb[j]) & (iota8 <= j), eq8, 0.0)
            tks.append(jnp.sum(accs[j] + corr, axis=0, keepdims=True))
        tmat_ref[pl.ds(base, _G), :] = jnp.concatenate(tks, axis=0)
        kap_ref[pl.ds(base, _G), :] = kq + jnp.uint32(1)
        return carry

    jax.lax.fori_loop(0, _NT, group, 0)

    acc = jnp.sum(jnp.log(tmat_ref[: _L, :]), axis=0, keepdims=True)
    col_loss = -sum_p + acc + jnp.float32(_L) * m
    partial = jnp.sum(col_loss) * jnp.float32(1.0 / _N)

    @pl.when(i == 0)
    def _():
        out_ref[...] = jnp.zeros_like(out_ref)

    out_ref[...] += jnp.full((1, 1), partial, jnp.float32)


def _sc_sort_key(t):
    s = lax.bitcast_convert_type(t, jnp.int32)
    m = lax.shift_right_arithmetic(s, 31)
    return lax.bitcast_convert_type(s ^ (m | jnp.int32(-2147483648)),
                                    jnp.uint32)


def _sc_log(x):
    i = lax.bitcast_convert_type(x, jnp.int32)
    y = (i.astype(jnp.float32) - jnp.float32(1064866805.0)) \
        * jnp.float32(8.262958405176314e-08)
    for _ in range(3):
        y = y + x * jnp.exp(-y) - jnp.float32(1.0)
    return y


def _sc_body(pt_hbm, tt_hbm, out_hbm, p_v, t_v, kap_v, e_v, acc_v):
    wid = lax.axis_index("s") * 2 + lax.axis_index("c")
    slab = wid // 8
    goff = (wid % 8) * 16

    pltpu.sync_copy(pt_hbm.at[slab], p_v)
    pltpu.sync_copy(tt_hbm.at[slab], t_v)

    def stats(l, c):
        mx, sp = c
        row = p_v[l, pl.ds(goff, 16)]
        return jnp.maximum(mx, row), sp + row

    m, sum_p = lax.fori_loop(
        0, _L, stats,
        (jnp.full((16,), -jnp.inf, jnp.float32),
         jnp.zeros((16,), jnp.float32)))

    def prep(l, c):
        kap_v[l] = _sc_sort_key(t_v[l, pl.ds(goff, 16)])
        e_v[l] = jnp.exp(p_v[l, pl.ds(goff, 16)] - m)
        return c

    lax.fori_loop(0, _L, prep, 0)

    def query(k, logsum):
        kb = kap_v[k] + jnp.uint32(1)
        kap_v[k] = kb
        kap_k = kb - jnp.uint32(1)
        tsum = jnp.zeros((16,), jnp.float32)
        for l in range(_L):
            tsum = tsum + jnp.where(kap_v[l] > kap_k, e_v[l], 0.0)
        return logsum + _sc_log(tsum)

    logsum = lax.fori_loop(0, _L, query, jnp.zeros((16,), jnp.float32))

    acc_v[0] = -sum_p + logsum + jnp.float32(_L) * m
    pltpu.sync_copy(acc_v, out_hbm.at[pl.ds(wid, 1), :])


@jax.jit
def kernel(y_pred, y_true):
    pt = y_pred.T
    tt = y_true.T
    tc_out = pl.pallas_call(
        _body,
        grid=(_TCB,),
        in_specs=[
            pl.BlockSpec((_L, _C), lambda i: (0, i)),
            pl.BlockSpec((_L, _C), lambda i: (0, i)),
        ],
        out_specs=pl.BlockSpec((1, 1), lambda i: (0, 0)),
        out_shape=jax.ShapeDtypeStruct((1, 1), jnp.float32),
        scratch_shapes=[
            pltpu.VMEM((_L, _C), jnp.uint32),
            pltpu.VMEM((_L, _C), jnp.float32),
            pltpu.VMEM((_L, _C), jnp.float32),
        ],
    )(pt, tt)

    tail_p = y_pred[_TCB * _C:].reshape(_NSLAB, 128, _L).transpose(0, 2, 1)
    tail_t = y_true[_TCB * _C:].reshape(_NSLAB, 128, _L).transpose(0, 2, 1)
    mesh = plsc.VectorSubcoreMesh(core_axis_name="c", subcore_axis_name="s")
    sc_partials = pl.kernel(
        _sc_body,
        mesh=mesh,
        out_type=jax.ShapeDtypeStruct((_NW, 16), jnp.float32),
        scratch_types=[
            pltpu.VMEM((_L, 128), jnp.float32),
            pltpu.VMEM((_L, 128), jnp.float32),
            pltpu.VMEM((_L, 16), jnp.uint32),
            pltpu.VMEM((_L, 16), jnp.float32),
            pltpu.VMEM((1, 16), jnp.float32),
        ],
    )(tail_p, tail_t)

    return tc_out[0, 0] + jnp.sum(sc_partials) * jnp.float32(1.0 / _N)


# speedup vs baseline: 6.2976x; 1.1293x over previous
import functools

import jax
import jax.numpy as jnp
from jax.experimental import pallas as pl
from jax.experimental.pallas import tpu as pltpu

_N = 4096
_L = 200
_C = 256
_G = 8
_NT = _L // _G


def _sort_key(t):
    t = jnp.where(t == 0.0, 0.0, t)
    s = jax.lax.bitcast_convert_type(t, jnp.int32)
    m = jax.lax.shift_right_arithmetic(s, jnp.int32(31))
    return jax.lax.bitcast_convert_type(s ^ (m | jnp.int32(-2147483648)),
                                        jnp.uint32)


def _body(pt_ref, tt_ref, out_ref, kap_ref, e_ref, tmat_ref):
    i = pl.program_id(0)
    p = pt_ref[...]
    t = tt_ref[...]
    m = jnp.max(p, axis=0, keepdims=True)
    e_ref[: _L, :] = jnp.exp(p - m)
    sum_p = jnp.sum(p, axis=0, keepdims=True)
    kap_ref[: _L, :] = _sort_key(t)

    iota8 = jax.lax.broadcasted_iota(jnp.uint32, (_G, _C), 0)

    def group(g, carry):
        base = g * _G
        kq = kap_ref[pl.ds(base, _G), :]
        eq8 = e_ref[pl.ds(base, _G), :]

        kqb = [jnp.broadcast_to(kq[j:j + 1, :], (_G, _C)) for j in range(_G)]

        accs = [jnp.zeros((_G, _C), jnp.float32) for _ in range(_G)]
        for tile in range(_NT):
            lhs = kap_ref[tile * _G:(tile + 1) * _G, :]
            et = e_ref[tile * _G:(tile + 1) * _G, :]
            for j in range(_G):
                accs[j] = accs[j] + jnp.where(lhs > kqb[j], et, 0.0)

        tks = []
        for j in range(_G):
            corr = jnp.where((kq == kqb[j]) & (iota8 <= j), eq8, 0.0)
            tks.append(jnp.sum(accs[j] + corr, axis=0, keepdims=True))
        tmat_ref[pl.ds(base, _G), :] = jnp.concatenate(tks, axis=0)

        kap_ref[pl.ds(base, _G), :] = kq + jnp.uint32(1)
        return carry

    jax.lax.fori_loop(0, _NT, group, 0)

    acc = jnp.sum(jnp.log(tmat_ref[: _L, :]), axis=0, keepdims=True)
    col_loss = -sum_p + acc + jnp.float32(_L) * m
    partial = jnp.sum(col_loss) * jnp.float32(1.0 / _N)

    @pl.when(i == 0)
    def _():
        out_ref[...] = jnp.zeros_like(out_ref)

    out_ref[...] += jnp.full((1, 1), partial, jnp.float32)


@jax.jit
def kernel(y_pred, y_true):
    pt = y_pred.T
    tt = y_true.T
    out = pl.pallas_call(
        _body,
        grid=(_N // _C,),
        in_specs=[
            pl.BlockSpec((_L, _C), lambda i: (0, i)),
            pl.BlockSpec((_L, _C), lambda i: (0, i)),
        ],
        out_specs=pl.BlockSpec((1, 1), lambda i: (0, 0)),
        out_shape=jax.ShapeDtypeStruct((1, 1), jnp.float32),
        scratch_shapes=[
            pltpu.VMEM((_L, _C), jnp.uint32),
            pltpu.VMEM((_L, _C), jnp.float32),
            pltpu.VMEM((_L, _C), jnp.float32),
        ],
    )(pt, tt)
    return out[0, 0]
